# baseline (speedup 1.0000x reference)
import jax
import jax.numpy as jnp
from jax import lax
from jax.experimental import pallas as pl
from jax.experimental.pallas import tpu as pltpu
from jax.experimental.pallas import tpu_sc as plsc

L = 16


def _sc_add(x2d, emb):
    R_TOTAL, D = x2d.shape
    NC, NS = 2, 16
    NW = NC * NS
    rows_per_w = R_TOTAL // NW
    CH = 16
    n_chunks = rows_per_w // CH
    rows_per_seg = R_TOTAL // emb.shape[0]
    d_steps = D // L

    mesh = plsc.VectorSubcoreMesh(core_axis_name="c", subcore_axis_name="s")

    def body(x_hbm, emb_hbm, out_hbm, emb_v, buf, sem):
        wid = lax.axis_index("s") * NC + lax.axis_index("c")
        base = wid * rows_per_w
        seg = base // rows_per_seg
        pltpu.sync_copy(emb_hbm.at[seg], emb_v)

        def chunk_body(i, _):
            row0 = base + i * CH
            pltpu.async_copy(x_hbm.at[pl.ds(row0, CH)], buf, sem).wait()

            def d_body(j, _):
                off = pl.multiple_of(j * L, L)
                e = emb_v[pl.ds(off, L)]

                def r_body(r, _):
                    buf[r, pl.ds(off, L)] += e
                    return 0

                lax.fori_loop(0, CH, r_body, 0)
                return 0

            lax.fori_loop(0, d_steps, d_body, 0)
            pltpu.async_copy(buf, out_hbm.at[pl.ds(row0, CH)], sem).wait()
            return 0

        lax.fori_loop(0, n_chunks, chunk_body, 0)

    return pl.kernel(
        body,
        out_type=jax.ShapeDtypeStruct((R_TOTAL, D), jnp.float32),
        mesh=mesh,
        scratch_types=[
            pltpu.VMEM((D,), jnp.float32),
            pltpu.VMEM((CH, D), jnp.float32),
            pltpu.SemaphoreType.DMA,
        ],
    )(x2d, emb)


def kernel(x, emb):
    S, B, D = x.shape
    out = _sc_add(x.reshape(S * B, D), emb)
    return out.reshape(S, B, D)


# speedup vs baseline: 1.8525x; 1.8525x over previous
import jax
import jax.numpy as jnp
from jax import lax
from jax.experimental import pallas as pl
from jax.experimental.pallas import tpu as pltpu
from jax.experimental.pallas import tpu_sc as plsc

L = 16


def _sc_add(x2d, emb):
    R_TOTAL, D = x2d.shape
    NC, NS = 2, 16
    NW = NC * NS
    rows_per_w = R_TOTAL // NW
    CH = 16
    n_chunks = rows_per_w // CH
    NBUF = 3
    LOOKAHEAD = 2
    rows_per_seg = R_TOTAL // emb.shape[0]
    d_steps = D // L

    mesh = plsc.VectorSubcoreMesh(core_axis_name="c", subcore_axis_name="s")

    def body(x_hbm, emb_hbm, out_hbm, emb_v, b0, b1, b2, *sems):
        bufs = (b0, b1, b2)
        in_sems = sems[:NBUF]
        out_sems = sems[NBUF:]
        wid = lax.axis_index("s") * NC + lax.axis_index("c")
        base = wid * rows_per_w
        seg = base // rows_per_seg
        pltpu.sync_copy(emb_hbm.at[seg], emb_v)

        def compute(buf):
            def d_body(j, _):
                off = pl.multiple_of(j * L, L)
                e = emb_v[pl.ds(off, L)]
                for r in range(CH):
                    buf[r, pl.ds(off, L)] += e
                return 0

            lax.fori_loop(0, d_steps, d_body, 0)

        def in_copy(i, b):
            return pltpu.async_copy(
                x_hbm.at[pl.ds(base + i * CH, CH)], bufs[b], in_sems[b])

        def out_copy(i, b):
            return pltpu.async_copy(
                bufs[b], out_hbm.at[pl.ds(base + i * CH, CH)], out_sems[b])

        in_flight = {}
        out_flight = {}
        for i in range(min(LOOKAHEAD, n_chunks)):
            in_flight[i] = in_copy(i, i % NBUF)
        for i in range(n_chunks):
            b = i % NBUF
            j = i + LOOKAHEAD
            if j < n_chunks:
                if j >= NBUF:
                    out_flight.pop(j - NBUF).wait()
                in_flight[j] = in_copy(j, j % NBUF)
            in_flight.pop(i).wait()
            compute(bufs[b])
            out_flight[i] = out_copy(i, b)
        for c in out_flight.values():
            c.wait()

    return pl.kernel(
        body,
        out_type=jax.ShapeDtypeStruct((R_TOTAL, D), jnp.float32),
        mesh=mesh,
        scratch_types=[
            pltpu.VMEM((D,), jnp.float32),
            pltpu.VMEM((CH, D), jnp.float32),
            pltpu.VMEM((CH, D), jnp.float32),
            pltpu.VMEM((CH, D), jnp.float32),
            pltpu.SemaphoreType.DMA,
            pltpu.SemaphoreType.DMA,
            pltpu.SemaphoreType.DMA,
            pltpu.SemaphoreType.DMA,
            pltpu.SemaphoreType.DMA,
            pltpu.SemaphoreType.DMA,
        ],
    )(x2d, emb)


def kernel(x, emb):
    S, B, D = x.shape
    out = _sc_add(x.reshape(S * B, D), emb)
    return out.reshape(S, B, D)


# speedup vs baseline: 6.6405x; 3.5845x over previous
import jax
import jax.numpy as jnp
from jax import lax
from jax.experimental import pallas as pl
from jax.experimental.pallas import tpu as pltpu
from jax.experimental.pallas import tpu_sc as plsc

L = 16


def kernel(x, emb):
    S, B, D = x.shape
    n_src = emb.shape[0]
    NC, NS = 2, 16
    NW = NC * NS
    rows_per_w = S // NW
    CH = 8
    n_chunks = rows_per_w // CH
    NBUF = 3
    LOOKAHEAD = 2
    rows_per_seg = S // n_src
    d_steps = D // L

    mesh = plsc.VectorSubcoreMesh(core_axis_name="c", subcore_axis_name="s")

    def body(x_hbm, emb_hbm, out_hbm, emb_v, b0, b1, b2, *sems):
        bufs = (b0, b1, b2)
        in_sems = sems[:NBUF]
        out_sems = sems[NBUF:]
        wid = lax.axis_index("s") * NC + lax.axis_index("c")
        base = wid * rows_per_w
        seg = base // rows_per_seg
        pltpu.sync_copy(emb_hbm, emb_v)

        def compute(buf):
            def d_body(j, _):
                off = pl.multiple_of(j * L, L)
                e = emb_v[seg, pl.ds(off, L)]
                for r in range(CH):
                    for b in range(B):
                        buf[r, b, pl.ds(off, L)] += e
                return 0

            lax.fori_loop(0, d_steps, d_body, 0)

        def in_copy(i, b):
            return pltpu.async_copy(
                x_hbm.at[pl.ds(base + i * CH, CH)], bufs[b], in_sems[b])

        def out_copy(i, b):
            return pltpu.async_copy(
                bufs[b], out_hbm.at[pl.ds(base + i * CH, CH)], out_sems[b])

        in_flight = {}
        out_flight = {}
        for i in range(min(LOOKAHEAD, n_chunks)):
            in_flight[i] = in_copy(i, i % NBUF)
        for i in range(n_chunks):
            b = i % NBUF
            j = i + LOOKAHEAD
            if j < n_chunks:
                if j >= NBUF:
                    out_flight.pop(j - NBUF).wait()
                in_flight[j] = in_copy(j, j % NBUF)
            in_flight.pop(i).wait()
            compute(bufs[b])
            out_flight[i] = out_copy(i, b)
        for c in out_flight.values():
            c.wait()

    return pl.kernel(
        body,
        out_type=jax.ShapeDtypeStruct((S, B, D), jnp.float32),
        mesh=mesh,
        scratch_types=[
            pltpu.VMEM((n_src, D), jnp.float32),
            pltpu.VMEM((CH, B, D), jnp.float32),
            pltpu.VMEM((CH, B, D), jnp.float32),
            pltpu.VMEM((CH, B, D), jnp.float32),
            pltpu.SemaphoreType.DMA,
            pltpu.SemaphoreType.DMA,
            pltpu.SemaphoreType.DMA,
            pltpu.SemaphoreType.DMA,
            pltpu.SemaphoreType.DMA,
            pltpu.SemaphoreType.DMA,
        ],
    )(x, emb)
